# Initial kernel scaffold; baseline (speedup 1.0000x reference)
import functools

import jax
import jax.numpy as jnp
from jax import lax
from jax.experimental import pallas as pl
from jax.experimental.pallas import tpu as pltpu
from jax.experimental.pallas import tpu_sc as plsc

N = 10000
C = 128
K = 16
B = 4
NPB = N // B
NPB_PAD = 2560
NPAD = 10240

NW = 32
NPW = NPAD // NW
CH = 8
NCHUNK = NPW // CH

RT = 256
MT = 1024

_BIGI = jnp.int32(2 ** 30)
_INF = jnp.float32(jnp.inf)


def _knn_body(pr_ref, pc_ref, out_ref):
    s = pl.program_id(0)
    r = pl.program_id(1)
    pr = pr_ref[0]
    pc = pc_ref[0]
    dot = jnp.dot(pr, pc, preferred_element_type=jnp.float32)
    sqr = jnp.sum(pr * pr, axis=1, keepdims=True)
    sqc = jnp.sum(pc * pc, axis=0, keepdims=True)
    d = (sqr + sqc) - 2.0 * dot

    cols = lax.broadcasted_iota(jnp.int32, (RT, NPB_PAD), 1)
    rows = r * RT + lax.broadcasted_iota(jnp.int32, (RT, NPB_PAD), 0)
    d = d + jnp.where(cols == rows, jnp.float32(1e10), jnp.float32(0.0))
    d = jnp.where(cols >= NPB, _INF, d)

    lane_k = lax.broadcasted_iota(jnp.int32, (RT, K), 1)

    def step(t, carry):
        d_c, acc = carry
        m = jnp.min(d_c, axis=1, keepdims=True)
        eq = d_c == m
        idx_t = jnp.min(jnp.where(eq, cols, _BIGI), axis=1, keepdims=True)
        acc = jnp.where(lane_k == t, jnp.broadcast_to(idx_t, (RT, K)), acc)
        d_c = jnp.where(cols == idx_t, _INF, d_c)
        return d_c, acc

    _, acc = lax.fori_loop(0, K, step, (d, jnp.zeros((RT, K), jnp.int32)))
    out_ref[0] = acc + s * NPB


def _knn(pr, pcT):
    return pl.pallas_call(
        _knn_body,
        grid=(B, NPB_PAD // RT),
        in_specs=[
            pl.BlockSpec((1, RT, 8), lambda s, r: (s, r, 0)),
            pl.BlockSpec((1, 8, NPB_PAD), lambda s, r: (s, 0, 0)),
        ],
        out_specs=pl.BlockSpec((1, RT, K), lambda s, r: (s, r, 0)),
        out_shape=jax.ShapeDtypeStruct((B, NPB_PAD, K), jnp.int32),
        compiler_params=pltpu.CompilerParams(
            dimension_semantics=("parallel", "parallel")),
    )(pr, pcT)


def _mm_body(x_ref, w_ref, b_ref, u_ref, v_ref):
    xb = x_ref[...]
    wa = w_ref[0:C, :]
    wb = w_ref[C:2 * C, :]
    u_ref[...] = jnp.dot(xb, wa - wb, preferred_element_type=jnp.float32) + b_ref[...]
    v_ref[...] = jnp.dot(xb, wb, preferred_element_type=jnp.float32)


def _mm(x, w, b):
    return pl.pallas_call(
        _mm_body,
        grid=(NPAD // MT,),
        in_specs=[
            pl.BlockSpec((MT, C), lambda i: (i, 0)),
            pl.BlockSpec((2 * C, C), lambda i: (0, 0)),
            pl.BlockSpec((1, C), lambda i: (0, 0)),
        ],
        out_specs=[
            pl.BlockSpec((MT, C), lambda i: (i, 0)),
            pl.BlockSpec((MT, C), lambda i: (i, 0)),
        ],
        out_shape=[
            jax.ShapeDtypeStruct((NPAD, C), jnp.float32),
            jax.ShapeDtypeStruct((NPAD, C), jnp.float32),
        ],
        compiler_params=pltpu.CompilerParams(
            dimension_semantics=("parallel",)),
    )(x, w, b.reshape(1, C))


def _make_gather_max(with_res):
    mesh = plsc.VectorSubcoreMesh(core_axis_name="c", subcore_axis_name="s")
    scratch = [
        pltpu.VMEM((CH * K,), jnp.int32),
        pltpu.VMEM((CH * K, C), jnp.float32),
        pltpu.VMEM((CH, C), jnp.float32),
        pltpu.VMEM((CH, C), jnp.float32),
    ]
    if with_res:
        scratch.append(pltpu.VMEM((CH, C), jnp.float32))
    scratch.append(pltpu.SemaphoreType.DMA)

    @functools.partial(
        pl.kernel,
        out_type=jax.ShapeDtypeStruct((NPAD, C), jnp.float32),
        mesh=mesh,
        scratch_types=scratch,
    )
    def k(*refs):
        if with_res:
            (idx_hbm, u_hbm, v_hbm, res_hbm, out_hbm,
             idx_v, rows_v, u_v, o_v, r_v, sem) = refs
        else:
            idx_hbm, u_hbm, v_hbm, out_hbm, idx_v, rows_v, u_v, o_v, sem = refs
            res_hbm = r_v = None
        wid = lax.axis_index("s") * 2 + lax.axis_index("c")
        base = wid * NPW

        def chunk(ci, carry):
            nb = base + ci * CH
            pltpu.sync_copy(idx_hbm.at[pl.ds(nb * K, CH * K)], idx_v)
            cp = pltpu.async_copy(v_hbm.at[idx_v], rows_v, sem)
            pltpu.sync_copy(u_hbm.at[pl.ds(nb, CH)], u_v)
            if with_res:
                pltpu.sync_copy(res_hbm.at[pl.ds(nb, CH)], r_v)
            cp.wait()
            for n in range(CH):
                for g in range(C // 16):
                    sl = pl.ds(g * 16, 16)
                    acc = rows_v[n * K, sl]
                    for j in range(1, K):
                        acc = jnp.maximum(acc, rows_v[n * K + j, sl])
                    val = jnp.maximum(acc + u_v[n, sl], jnp.float32(0.0))
                    if with_res:
                        val = val + r_v[n, sl]
                    o_v[n, sl] = val
            pltpu.sync_copy(o_v, out_hbm.at[pl.ds(nb, CH)])
            return carry

        lax.fori_loop(0, NCHUNK, chunk, 0)

    return k


_gm_nores = _make_gather_max(False)
_gm_res = _make_gather_max(True)


def kernel(pillar_features, voxel_coords, W1, b1, W2, b2):
    pos3 = voxel_coords[:, 1:4].reshape(B, NPB, 3)
    pr = jnp.pad(pos3, ((0, 0), (0, NPB_PAD - NPB), (0, 5)))
    pcT = jnp.transpose(pr, (0, 2, 1))
    idx_all = _knn(pr, pcT)
    idx = idx_all[:, :NPB, :].reshape(N, K)
    idx_flat = jnp.pad(idx, ((0, NPAD - N), (0, 0))).reshape(-1)

    xp = jnp.pad(pillar_features, ((0, NPAD - N), (0, 0)))
    u1, v1 = _mm(xp, W1, b1)
    f1 = _gm_nores(idx_flat, u1, v1)
    u2, v2 = _mm(f1, W2, b2)
    out = _gm_res(idx_flat, u2, v2, xp)
    return out[:N]


# speedup vs baseline: 5.1238x; 5.1238x over previous
import functools

import jax
import jax.numpy as jnp
from jax import lax
from jax.experimental import pallas as pl
from jax.experimental.pallas import tpu as pltpu
from jax.experimental.pallas import tpu_sc as plsc

N = 10000
C = 128
K = 16
B = 4
NPB = N // B
NPB_PAD = 2560
NPAD = 10240

NW = 32
NPW = NPAD // NW
CH = 8
NCHUNK = NPW // CH

RT = 256
MT = 1024

_BIGI = 2 ** 30
_INF = float("inf")


def _knn_body(pr_ref, pc_ref, out_ref):
    s = pl.program_id(0)
    r = pl.program_id(1)
    pr = pr_ref[0]
    pc = pc_ref[0]
    dot = jnp.dot(pr, pc, preferred_element_type=jnp.float32)
    sqr = jnp.sum(pr * pr, axis=1, keepdims=True)
    sqc = jnp.sum(pc * pc, axis=0, keepdims=True)
    d = (sqr + sqc) - 2.0 * dot

    cols = lax.broadcasted_iota(jnp.int32, (RT, NPB_PAD), 1)
    rows = r * RT + lax.broadcasted_iota(jnp.int32, (RT, NPB_PAD), 0)
    d = d + jnp.where(cols == rows, jnp.float32(1e10), jnp.float32(0.0))
    d = jnp.where(cols >= NPB, jnp.float32(_INF), d)

    lane_k = lax.broadcasted_iota(jnp.int32, (RT, K), 1)

    def step(t, carry):
        d_c, acc = carry
        m = jnp.min(d_c, axis=1, keepdims=True)
        eq = d_c == m
        idx_t = jnp.min(jnp.where(eq, cols, jnp.int32(_BIGI)), axis=1, keepdims=True)
        acc = jnp.where(lane_k == t, jnp.broadcast_to(idx_t, (RT, K)), acc)
        d_c = jnp.where(cols == idx_t, jnp.float32(_INF), d_c)
        return d_c, acc

    _, acc = lax.fori_loop(0, K, step, (d, jnp.zeros((RT, K), jnp.int32)))
    out_ref[0] = acc + s * NPB


def _knn(pr, pcT):
    return pl.pallas_call(
        _knn_body,
        grid=(B, NPB_PAD // RT),
        in_specs=[
            pl.BlockSpec((1, RT, 8), lambda s, r: (s, r, 0)),
            pl.BlockSpec((1, 8, NPB_PAD), lambda s, r: (s, 0, 0)),
        ],
        out_specs=pl.BlockSpec((1, RT, K), lambda s, r: (s, r, 0)),
        out_shape=jax.ShapeDtypeStruct((B, NPB_PAD, K), jnp.int32),
        compiler_params=pltpu.CompilerParams(
            dimension_semantics=("parallel", "parallel")),
    )(pr, pcT)


def _mm_body(x_ref, w_ref, b_ref, u_ref, v_ref):
    xb = x_ref[...]
    wa = w_ref[0:C, :]
    wb = w_ref[C:2 * C, :]
    u_ref[...] = jnp.dot(xb, wa - wb, preferred_element_type=jnp.float32) + b_ref[...]
    v_ref[...] = jnp.dot(xb, wb, preferred_element_type=jnp.float32)


def _mm(x, w, b):
    return pl.pallas_call(
        _mm_body,
        grid=(NPAD // MT,),
        in_specs=[
            pl.BlockSpec((MT, C), lambda i: (i, 0)),
            pl.BlockSpec((2 * C, C), lambda i: (0, 0)),
            pl.BlockSpec((1, C), lambda i: (0, 0)),
        ],
        out_specs=[
            pl.BlockSpec((MT, C), lambda i: (i, 0)),
            pl.BlockSpec((MT, C), lambda i: (i, 0)),
        ],
        out_shape=[
            jax.ShapeDtypeStruct((NPAD, C), jnp.float32),
            jax.ShapeDtypeStruct((NPAD, C), jnp.float32),
        ],
        compiler_params=pltpu.CompilerParams(
            dimension_semantics=("parallel",)),
    )(x, w, b.reshape(1, C))


def _make_gather_max(with_res):
    mesh = plsc.VectorSubcoreMesh(core_axis_name="c", subcore_axis_name="s",
                                  num_cores=2, num_subcores=16)
    scratch = [
        pltpu.VMEM((CH * K,), jnp.int32),
        pltpu.VMEM((CH * K, C), jnp.float32),
        pltpu.VMEM((CH, C), jnp.float32),
        pltpu.VMEM((CH, C), jnp.float32),
    ]
    if with_res:
        scratch.append(pltpu.VMEM((CH, C), jnp.float32))
    scratch.append(pltpu.SemaphoreType.DMA)

    @functools.partial(
        pl.kernel,
        out_type=jax.ShapeDtypeStruct((NPAD, C), jnp.float32),
        mesh=mesh,
        scratch_types=scratch,
    )
    def k(*refs):
        if with_res:
            (idx_hbm, u_hbm, v_hbm, res_hbm, out_hbm,
             idx_v, rows_v, u_v, o_v, r_v, sem) = refs
        else:
            idx_hbm, u_hbm, v_hbm, out_hbm, idx_v, rows_v, u_v, o_v, sem = refs
            res_hbm = r_v = None
        wid = lax.axis_index("s") * 2 + lax.axis_index("c")
        base = wid * NPW

        def chunk(ci, carry):
            nb = base + ci * CH
            pltpu.sync_copy(idx_hbm.at[pl.ds(nb * K, CH * K)], idx_v)
            cp = pltpu.async_copy(v_hbm.at[idx_v], rows_v, sem)
            pltpu.sync_copy(u_hbm.at[pl.ds(nb, CH)], u_v)
            if with_res:
                pltpu.sync_copy(res_hbm.at[pl.ds(nb, CH)], r_v)
            cp.wait()
            for n in range(CH):
                for g in range(C // 16):
                    sl = pl.ds(g * 16, 16)
                    acc = rows_v[n * K, sl]
                    for j in range(1, K):
                        acc = jnp.maximum(acc, rows_v[n * K + j, sl])
                    val = jnp.maximum(acc + u_v[n, sl], jnp.float32(0.0))
                    if with_res:
                        val = val + r_v[n, sl]
                    o_v[n, sl] = val
            pltpu.sync_copy(o_v, out_hbm.at[pl.ds(nb, CH)])
            return carry

        lax.fori_loop(0, NCHUNK, chunk, 0)

    return k


@functools.cache
def _gather_max(with_res):
    return _make_gather_max(with_res)


def kernel(pillar_features, voxel_coords, W1, b1, W2, b2):
    pos3 = voxel_coords[:, 1:4].reshape(B, NPB, 3)
    pr = jnp.pad(pos3, ((0, 0), (0, NPB_PAD - NPB), (0, 5)))
    pcT = jnp.transpose(pr, (0, 2, 1))
    idx_all = _knn(pr, pcT)
    idx = idx_all[:, :NPB, :].reshape(N, K)
    idx_flat = jnp.pad(idx, ((0, NPAD - N), (0, 0))).reshape(-1)

    xp = jnp.pad(pillar_features, ((0, NPAD - N), (0, 0)))
    u1, v1 = _mm(xp, W1, b1)
    f1 = _gather_max(False)(idx_flat, u1, v1)
    u2, v2 = _mm(f1, W2, b2)
    out = _gather_max(True)(idx_flat, u2, v2, xp)
    return out[:N]


# speedup vs baseline: 5.8842x; 1.1484x over previous
import functools

import jax
import jax.numpy as jnp
from jax import lax
from jax.experimental import pallas as pl
from jax.experimental.pallas import tpu as pltpu
from jax.experimental.pallas import tpu_sc as plsc

N = 10000
C = 128
K = 16
B = 4
NPB = N // B
NPB_PAD = 2560
NPAD = 10240

NW = 32
NPW = NPAD // NW
CH = 8
NCHUNK = NPW // CH

RT = 256
MT = 1024

_BIGI = 2 ** 30
_INF = float("inf")


def _knn_body(pr_ref, pc_ref, out_ref):
    s = pl.program_id(0)
    r = pl.program_id(1)
    pr = pr_ref[0]
    pc = pc_ref[0]
    dot = jnp.dot(pr, pc, preferred_element_type=jnp.float32)
    sqr = jnp.sum(pr * pr, axis=1, keepdims=True)
    sqc = jnp.sum(pc * pc, axis=0, keepdims=True)
    d = (sqr + sqc) - 2.0 * dot

    cols = lax.broadcasted_iota(jnp.int32, (RT, NPB_PAD), 1)
    rows = r * RT + lax.broadcasted_iota(jnp.int32, (RT, NPB_PAD), 0)
    d = d + jnp.where(cols == rows, jnp.float32(1e10), jnp.float32(0.0))
    d = jnp.where(cols >= NPB, jnp.float32(_INF), d)

    lane_k = lax.broadcasted_iota(jnp.int32, (RT, K), 1)

    def step(t, carry):
        d_c, acc = carry
        m = jnp.min(d_c, axis=1, keepdims=True)
        eq = d_c == m
        idx_t = jnp.min(jnp.where(eq, cols, jnp.int32(_BIGI)), axis=1, keepdims=True)
        acc = jnp.where(lane_k == t, jnp.broadcast_to(idx_t, (RT, K)), acc)
        d_c = jnp.where(cols == idx_t, jnp.float32(_INF), d_c)
        return d_c, acc

    _, acc = lax.fori_loop(0, K, step, (d, jnp.zeros((RT, K), jnp.int32)))
    out_ref[0] = acc + s * NPB


def _knn(pr, pcT):
    return pl.pallas_call(
        _knn_body,
        grid=(B, NPB_PAD // RT),
        in_specs=[
            pl.BlockSpec((1, RT, 8), lambda s, r: (s, r, 0)),
            pl.BlockSpec((1, 8, NPB_PAD), lambda s, r: (s, 0, 0)),
        ],
        out_specs=pl.BlockSpec((1, RT, K), lambda s, r: (s, r, 0)),
        out_shape=jax.ShapeDtypeStruct((B, NPB_PAD, K), jnp.int32),
        compiler_params=pltpu.CompilerParams(
            dimension_semantics=("parallel", "parallel")),
    )(pr, pcT)


def _mm_body(x_ref, w_ref, b_ref, u_ref, v_ref):
    xb = x_ref[...]
    wa = w_ref[0:C, :]
    wb = w_ref[C:2 * C, :]
    u_ref[...] = jnp.dot(xb, wa - wb, preferred_element_type=jnp.float32) + b_ref[...]
    v_ref[...] = jnp.dot(xb, wb, preferred_element_type=jnp.float32)


def _mm(x, w, b):
    return pl.pallas_call(
        _mm_body,
        grid=(NPAD // MT,),
        in_specs=[
            pl.BlockSpec((MT, C), lambda i: (i, 0)),
            pl.BlockSpec((2 * C, C), lambda i: (0, 0)),
            pl.BlockSpec((1, C), lambda i: (0, 0)),
        ],
        out_specs=[
            pl.BlockSpec((MT, C), lambda i: (i, 0)),
            pl.BlockSpec((MT, C), lambda i: (i, 0)),
        ],
        out_shape=[
            jax.ShapeDtypeStruct((NPAD, C), jnp.float32),
            jax.ShapeDtypeStruct((NPAD, C), jnp.float32),
        ],
        compiler_params=pltpu.CompilerParams(
            dimension_semantics=("parallel",)),
    )(x, w, b.reshape(1, C))


CHN = 16
NCH = NPW // CHN
IDXROWS = NPW * K // 128


def _make_gather_max(with_res):
    mesh = plsc.VectorSubcoreMesh(core_axis_name="c", subcore_axis_name="s",
                                  num_cores=2, num_subcores=16)
    scratch = [
        pltpu.VMEM((IDXROWS, 128), jnp.int32),
        pltpu.VMEM((CHN * K, C), jnp.float32),
        pltpu.VMEM((CHN * K, C), jnp.float32),
        pltpu.VMEM((CHN, C), jnp.float32),
        pltpu.VMEM((CHN, C), jnp.float32),
        pltpu.VMEM((NPW, C), jnp.float32),
    ]
    if with_res:
        scratch.append(pltpu.VMEM((CHN, C), jnp.float32))
        scratch.append(pltpu.VMEM((CHN, C), jnp.float32))
    scratch.append(pltpu.SemaphoreType.DMA)
    scratch.append(pltpu.SemaphoreType.DMA)

    @functools.partial(
        pl.kernel,
        out_type=jax.ShapeDtypeStruct((NPAD, C), jnp.float32),
        mesh=mesh,
        scratch_types=scratch,
    )
    def k(*refs):
        if with_res:
            (idx_hbm, u_hbm, v_hbm, res_hbm, out_hbm,
             idx_v, rows0, rows1, u0, u1, o_all, r0, r1, sem0, sem1) = refs
            r_v = (r0, r1)
        else:
            (idx_hbm, u_hbm, v_hbm, out_hbm,
             idx_v, rows0, rows1, u0, u1, o_all, sem0, sem1) = refs
            res_hbm = None
            r_v = (None, None)
        rows = (rows0, rows1)
        u_v = (u0, u1)
        sems = (sem0, sem1)
        wid = lax.axis_index("s") * 2 + lax.axis_index("c")
        base = wid * NPW

        pltpu.sync_copy(idx_hbm.at[pl.ds(wid * IDXROWS, IDXROWS)], idx_v)

        def issue(c, b):
            nb = base + c * CHN
            pltpu.async_copy(v_hbm.at[idx_v.at[2 * c]],
                             rows[b].at[pl.ds(0, 128)], sems[b])
            pltpu.async_copy(v_hbm.at[idx_v.at[2 * c + 1]],
                             rows[b].at[pl.ds(128, 128)], sems[b])
            pltpu.async_copy(u_hbm.at[pl.ds(nb, CHN)], u_v[b], sems[b])
            if with_res:
                pltpu.async_copy(res_hbm.at[pl.ds(nb, CHN)], r_v[b], sems[b])

        def drain(c, b):
            nb = base + c * CHN
            pltpu.make_async_copy(v_hbm.at[pl.ds(0, CHN * K)], rows[b],
                                  sems[b]).wait()
            pltpu.make_async_copy(u_hbm.at[pl.ds(nb, CHN)], u_v[b],
                                  sems[b]).wait()
            if with_res:
                pltpu.make_async_copy(res_hbm.at[pl.ds(nb, CHN)], r_v[b],
                                      sems[b]).wait()

        issue(0, 0)

        @pl.loop(0, NCH, step=2)
        def _pair(cbase):
            for b in range(2):
                c = cbase + b
                @pl.when(c + 1 < NCH)
                def _():
                    issue(c + 1, 1 - b)
                drain(c, b)

                def node(n, carry):
                    for g in range(C // 16):
                        sl = pl.ds(g * 16, 16)
                        acc = rows[b][n * K, sl]
                        for j in range(1, K):
                            acc = jnp.maximum(acc, rows[b][n * K + j, sl])
                        val = jnp.maximum(acc + u_v[b][n, sl], jnp.float32(0.0))
                        if with_res:
                            val = val + r_v[b][n, sl]
                        o_all[c * CHN + n, sl] = val
                    return carry
                lax.fori_loop(0, CHN, node, 0)

        pltpu.sync_copy(o_all, out_hbm.at[pl.ds(base, NPW)])

    return k


@functools.cache
def _gather_max(with_res):
    return _make_gather_max(with_res)


def kernel(pillar_features, voxel_coords, W1, b1, W2, b2):
    pos3 = voxel_coords[:, 1:4].reshape(B, NPB, 3)
    pr = jnp.pad(pos3, ((0, 0), (0, NPB_PAD - NPB), (0, 5)))
    pcT = jnp.transpose(pr, (0, 2, 1))
    idx_all = _knn(pr, pcT)
    idx = idx_all[:, :NPB, :].reshape(N, K)
    idx_flat = jnp.pad(idx, ((0, NPAD - N), (0, 0))).reshape(NPAD * K // 128, 128)

    xp = jnp.pad(pillar_features, ((0, NPAD - N), (0, 0)))
    u1, v1 = _mm(xp, W1, b1)
    f1 = _gather_max(False)(idx_flat, u1, v1)
    u2, v2 = _mm(f1, W2, b2)
    out = _gather_max(True)(idx_flat, u2, v2, xp)
    return out[:N]


# speedup vs baseline: 6.6887x; 1.1367x over previous
import functools

import jax
import jax.numpy as jnp
from jax import lax
from jax.experimental import pallas as pl
from jax.experimental.pallas import tpu as pltpu
from jax.experimental.pallas import tpu_sc as plsc

N = 10000
C = 128
K = 16
B = 4
NPB = N // B
NPB_PAD = 2560
H = 5120
NPAD = 2 * H

NW = 32
NPW = H // 16
CH = 8
NCHUNK = NPW // CH

RT = 256
MT = 1024

_BIGI = 2 ** 30
_INF = float("inf")


def _knn_body(pr_ref, pc_ref, out_ref):
    s = pl.program_id(0)
    r = pl.program_id(1)
    pr = pr_ref[0]
    pc = pc_ref[0]
    dot = jnp.dot(pr, pc, preferred_element_type=jnp.float32)
    sqr = jnp.sum(pr * pr, axis=1, keepdims=True)
    sqc = jnp.sum(pc * pc, axis=0, keepdims=True)
    d = (sqr + sqc) - 2.0 * dot

    cols = lax.broadcasted_iota(jnp.int32, (RT, NPB_PAD), 1)
    rows = r * RT + lax.broadcasted_iota(jnp.int32, (RT, NPB_PAD), 0)
    d = d + jnp.where(cols == rows, jnp.float32(1e10), jnp.float32(0.0))
    d = jnp.where(cols >= NPB, jnp.float32(_INF), d)

    lane_k = lax.broadcasted_iota(jnp.int32, (RT, K), 1)

    def step(t, carry):
        d_c, acc = carry
        m = jnp.min(d_c, axis=1, keepdims=True)
        eq = d_c == m
        idx_t = jnp.min(jnp.where(eq, cols, jnp.int32(_BIGI)), axis=1, keepdims=True)
        acc = jnp.where(lane_k == t, jnp.broadcast_to(idx_t, (RT, K)), acc)
        d_c = jnp.where(cols == idx_t, jnp.float32(_INF), d_c)
        return d_c, acc

    _, acc = lax.fori_loop(0, K, step, (d, jnp.zeros((RT, K), jnp.int32)))
    out_ref[0] = acc + (s * NPB + (s // 2) * (H - 2 * NPB))


def _knn(pr, pcT):
    return pl.pallas_call(
        _knn_body,
        grid=(B, NPB_PAD // RT),
        in_specs=[
            pl.BlockSpec((1, RT, 8), lambda s, r: (s, r, 0)),
            pl.BlockSpec((1, 8, NPB_PAD), lambda s, r: (s, 0, 0)),
        ],
        out_specs=pl.BlockSpec((1, RT, K), lambda s, r: (s, r, 0)),
        out_shape=jax.ShapeDtypeStruct((B, NPB_PAD, K), jnp.int32),
        compiler_params=pltpu.CompilerParams(
            dimension_semantics=("parallel", "parallel")),
    )(pr, pcT)


def _mm_body(x_ref, w_ref, b_ref, u_ref, v_ref):
    xb = x_ref[...]
    wa = w_ref[0:C, :]
    wb = w_ref[C:2 * C, :]
    u_ref[...] = jnp.dot(xb, wa - wb, preferred_element_type=jnp.float32) + b_ref[...]
    v_ref[...] = jnp.dot(xb, wb, preferred_element_type=jnp.float32)


def _mm(x, w, b):
    return pl.pallas_call(
        _mm_body,
        grid=(NPAD // MT,),
        in_specs=[
            pl.BlockSpec((MT, C), lambda i: (i, 0)),
            pl.BlockSpec((2 * C, C), lambda i: (0, 0)),
            pl.BlockSpec((1, C), lambda i: (0, 0)),
        ],
        out_specs=[
            pl.BlockSpec((MT, C), lambda i: (i, 0)),
            pl.BlockSpec((MT, C), lambda i: (i, 0)),
        ],
        out_shape=[
            jax.ShapeDtypeStruct((NPAD, C), jnp.float32),
            jax.ShapeDtypeStruct((NPAD, C), jnp.float32),
        ],
        compiler_params=pltpu.CompilerParams(
            dimension_semantics=("parallel",)),
    )(x, w, b.reshape(1, C))


CHN = 16
NCH = NPW // CHN
IDXROWS = NPW * K // 128


def _make_gather_max(with_res):
    mesh = plsc.VectorSubcoreMesh(core_axis_name="c", subcore_axis_name="s",
                                  num_cores=2, num_subcores=16)
    scratch = [
        pltpu.VMEM((IDXROWS, 128), jnp.int32),
        pltpu.VMEM((CHN * K, C), jnp.float32),
        pltpu.VMEM((CHN * K, C), jnp.float32),
        pltpu.VMEM((CHN, C), jnp.float32),
        pltpu.VMEM((CHN, C), jnp.float32),
        pltpu.VMEM((CHN, C), jnp.float32),
        pltpu.VMEM((CHN, C), jnp.float32),
    ]
    if with_res:
        scratch.append(pltpu.VMEM((CHN, C), jnp.float32))
        scratch.append(pltpu.VMEM((CHN, C), jnp.float32))
    scratch.append(pltpu.SemaphoreType.DMA)
    scratch.append(pltpu.SemaphoreType.DMA)
    scratch.append(pltpu.SemaphoreType.DMA)
    scratch.append(pltpu.SemaphoreType.DMA)

    @functools.partial(
        pl.kernel,
        out_type=jax.ShapeDtypeStruct((NPAD, C), jnp.float32),
        mesh=mesh,
        scratch_types=scratch,
    )
    def k(*refs):
        if with_res:
            (idx_hbm, u_hbm, v_hbm, res_hbm, out_hbm,
             idx_v, rows0, rows1, u0, u1, o0, o1, r0, r1,
             sem0, sem1, ssem0, ssem1) = refs
            r_v = (r0, r1)
        else:
            (idx_hbm, u_hbm, v_hbm, out_hbm,
             idx_v, rows0, rows1, u0, u1, o0, o1,
             sem0, sem1, ssem0, ssem1) = refs
            res_hbm = None
            r_v = (None, None)
        rows = (rows0, rows1)
        u_v = (u0, u1)
        o_v = (o0, o1)
        sems = (sem0, sem1)
        ssems = (ssem0, ssem1)
        cid = lax.axis_index("c")
        sid = lax.axis_index("s")
        base = cid * H + sid * NPW

        pltpu.sync_copy(idx_hbm.at[pl.ds(cid * (H * K // 128) + sid * IDXROWS, IDXROWS)], idx_v)

        def issue(c, b):
            nb = base + c * CHN
            pltpu.async_copy(v_hbm.at[idx_v.at[2 * c]],
                             rows[b].at[pl.ds(0, 128)], sems[b])
            pltpu.async_copy(v_hbm.at[idx_v.at[2 * c + 1]],
                             rows[b].at[pl.ds(128, 128)], sems[b])
            pltpu.async_copy(u_hbm.at[pl.ds(nb, CHN)], u_v[b], sems[b])
            if with_res:
                pltpu.async_copy(res_hbm.at[pl.ds(nb, CHN)], r_v[b], sems[b])

        def drain(c, b):
            nb = base + c * CHN
            pltpu.make_async_copy(v_hbm.at[pl.ds(0, CHN * K)], rows[b],
                                  sems[b]).wait()
            pltpu.make_async_copy(u_hbm.at[pl.ds(nb, CHN)], u_v[b],
                                  sems[b]).wait()
            if with_res:
                pltpu.make_async_copy(res_hbm.at[pl.ds(nb, CHN)], r_v[b],
                                      sems[b]).wait()

        issue(0, 0)

        def drain_store(c, b):
            pltpu.make_async_copy(o_v[b], out_hbm.at[pl.ds(base + c * CHN, CHN)],
                                  ssems[b]).wait()

        @pl.loop(0, NCH, step=2)
        def _pair(cbase):
            for b in range(2):
                c = cbase + b
                @pl.when(c + 1 < NCH)
                def _():
                    issue(c + 1, 1 - b)
                drain(c, b)
                @pl.when(c >= 2)
                def _():
                    drain_store(c - 2, b)

                def node(n, carry):
                    for g in range(C // 16):
                        sl = pl.ds(g * 16, 16)
                        acc = rows[b][n * K, sl]
                        for j in range(1, K):
                            acc = jnp.maximum(acc, rows[b][n * K + j, sl])
                        val = jnp.maximum(acc + u_v[b][n, sl], jnp.float32(0.0))
                        if with_res:
                            val = val + r_v[b][n, sl]
                        o_v[b][n, sl] = val
                    return carry
                lax.fori_loop(0, CHN, node, 0)
                pltpu.async_copy(o_v[b], out_hbm.at[pl.ds(base + c * CHN, CHN)],
                                 ssems[b])

        drain_store(NCH - 2, 0)
        drain_store(NCH - 1, 1)

    return k


@functools.cache
def _gather_max(with_res):
    return _make_gather_max(with_res)


def kernel(pillar_features, voxel_coords, W1, b1, W2, b2):
    pos3 = voxel_coords[:, 1:4].reshape(B, NPB, 3)
    pr = jnp.pad(pos3, ((0, 0), (0, NPB_PAD - NPB), (0, 5)))
    pcT = jnp.transpose(pr, (0, 2, 1))
    idx_all = _knn(pr, pcT)
    idx = idx_all[:, :NPB, :].reshape(2, 2 * NPB, K)
    idxp = jnp.pad(idx, ((0, 0), (0, H - 2 * NPB), (0, 0)))
    idx_local = idxp.reshape(NPAD * K // 128, 128)

    xp = jnp.pad(pillar_features.reshape(2, 2 * NPB, C),
                 ((0, 0), (0, H - 2 * NPB), (0, 0))).reshape(NPAD, C)
    u1, v1 = _mm(xp, W1, b1)
    f1 = _gather_max(False)(idx_local, u1, v1)
    u2, v2 = _mm(f1, W2, b2)
    out = _gather_max(True)(idx_local, u2, v2, xp)
    return out.reshape(2, H, C)[:, :2 * NPB].reshape(N, C)
